# initial kernel scaffold (unmeasured)
import jax
import jax.numpy as jnp
from jax import lax
from jax.experimental import pallas as pl
from jax.experimental.pallas import tpu as pltpu

N_DEV = 16
CAP = 63
BLK = CAP + 1
N_COLS = 256


def _a2a_pallas(send_buf):

    def body(send_ref, recv_ref, send_sems, recv_sems, local_sem):
        me = lax.axis_index("i")

        own = pltpu.make_async_copy(
            send_ref.at[me], recv_ref.at[me], local_sem
        )
        own.start()

        sends = []
        for k in range(1, N_DEV):
            tgt = lax.rem(me + k, N_DEV)
            rdma = pltpu.make_async_remote_copy(
                src_ref=send_ref.at[tgt],
                dst_ref=recv_ref.at[me],
                send_sem=send_sems.at[k],
                recv_sem=recv_sems.at[k],
                device_id=(tgt,),
                device_id_type=pl.DeviceIdType.MESH,
            )
            rdma.start()
            sends.append(rdma)

        for k in range(1, N_DEV):
            src = lax.rem(me - k + N_DEV, N_DEV)
            recv = pltpu.make_async_remote_copy(
                src_ref=send_ref.at[src],
                dst_ref=recv_ref.at[src],
                send_sem=send_sems.at[k],
                recv_sem=recv_sems.at[k],
                device_id=(src,),
                device_id_type=pl.DeviceIdType.MESH,
            )
            recv.wait_recv()

        for rdma in sends:
            rdma.wait_send()
        own.wait()

    return pl.pallas_call(
        body,
        out_shape=jax.ShapeDtypeStruct((N_DEV, BLK, N_COLS), jnp.bfloat16),
        in_specs=[pl.BlockSpec(memory_space=pltpu.VMEM)],
        out_specs=pl.BlockSpec(memory_space=pltpu.VMEM),
        scratch_shapes=[
            pltpu.SemaphoreType.DMA((N_DEV,)),
            pltpu.SemaphoreType.DMA((N_DEV,)),
            pltpu.SemaphoreType.DMA,
        ],
        compiler_params=pltpu.CompilerParams(collective_id=0),
    )(send_buf)


def kernel(x, dest):
    m = x.shape[0]
    ranks = jnp.arange(N_DEV, dtype=dest.dtype)

    order = jnp.argsort(dest, stable=True)
    counts = jnp.sum(dest[None, :] == ranks[:, None], axis=1)
    offs = jnp.concatenate(
        [jnp.zeros((1,), counts.dtype), jnp.cumsum(counts)[:-1]]
    )
    j = jnp.arange(CAP)
    idx = offs[:, None] + j[None, :]
    valid = j[None, :] < counts[:, None]
    rows = order[jnp.where(valid, idx, 0)]
    data = x[rows].astype(jnp.bfloat16)
    hdr = jnp.broadcast_to(
        counts.astype(jnp.bfloat16)[:, None, None], (N_DEV, 1, N_COLS)
    )
    send_buf = jnp.concatenate([hdr, data], axis=1)

    recv = _a2a_pallas(send_buf)

    rcnts = recv[:, 0, 0].astype(jnp.int32)
    roffs = jnp.concatenate(
        [jnp.zeros((1,), jnp.int32), jnp.cumsum(rcnts)[:-1]]
    )
    tgt = roffs[:, None] + j[None, :]
    rvalid = j[None, :] < rcnts[:, None]
    tgt = jnp.where(rvalid, tgt, m)
    out = jnp.zeros((m + 1, N_COLS), jnp.float32)
    out = out.at[tgt.reshape(-1)].set(
        recv[:, 1:, :].reshape(-1, N_COLS).astype(jnp.float32)
    )
    return out[:m]


# baseline (device time: 40628 ns/iter reference)
import jax
import jax.numpy as jnp
from jax import lax
from jax.experimental import pallas as pl
from jax.experimental.pallas import tpu as pltpu

N_DEV = 16
CAP = 63
BLK = CAP + 1
N_COLS = 256


def _a2a_pallas(send_buf):

    def body(send_ref, recv_ref, send_sems, recv_sems, local_sem):
        me = lax.axis_index("i")

        own = pltpu.make_async_copy(
            send_ref.at[me], recv_ref.at[me], local_sem
        )
        own.start()

        sends = []
        for k in range(1, N_DEV):
            tgt = lax.rem(me + k, N_DEV)
            rdma = pltpu.make_async_remote_copy(
                src_ref=send_ref.at[tgt],
                dst_ref=recv_ref.at[me],
                send_sem=send_sems.at[k],
                recv_sem=recv_sems.at[k],
                device_id=(tgt,),
                device_id_type=pl.DeviceIdType.MESH,
            )
            rdma.start()
            sends.append(rdma)

        for k in range(1, N_DEV):
            src = lax.rem(me - k + N_DEV, N_DEV)
            recv = pltpu.make_async_remote_copy(
                src_ref=send_ref.at[src],
                dst_ref=recv_ref.at[src],
                send_sem=send_sems.at[k],
                recv_sem=recv_sems.at[k],
                device_id=(src,),
                device_id_type=pl.DeviceIdType.MESH,
            )
            recv.wait_recv()

        for rdma in sends:
            rdma.wait_send()
        own.wait()

    return pl.pallas_call(
        body,
        out_shape=jax.ShapeDtypeStruct((N_DEV, BLK, N_COLS), jnp.bfloat16),
        in_specs=[pl.BlockSpec(memory_space=pltpu.VMEM)],
        out_specs=pl.BlockSpec(memory_space=pltpu.VMEM),
        scratch_shapes=[
            pltpu.SemaphoreType.DMA((N_DEV,)),
            pltpu.SemaphoreType.DMA((N_DEV,)),
            pltpu.SemaphoreType.DMA,
        ],
    )(send_buf)


def kernel(x, dest):
    m = x.shape[0]
    ranks = jnp.arange(N_DEV, dtype=dest.dtype)

    order = jnp.argsort(dest, stable=True)
    counts = jnp.sum(dest[None, :] == ranks[:, None], axis=1)
    offs = jnp.concatenate(
        [jnp.zeros((1,), counts.dtype), jnp.cumsum(counts)[:-1]]
    )
    j = jnp.arange(CAP)
    idx = offs[:, None] + j[None, :]
    valid = j[None, :] < counts[:, None]
    rows = order[jnp.where(valid, idx, 0)]
    data = x[rows].astype(jnp.bfloat16)
    hdr = jnp.broadcast_to(
        counts.astype(jnp.bfloat16)[:, None, None], (N_DEV, 1, N_COLS)
    )
    send_buf = jnp.concatenate([hdr, data], axis=1)

    recv = _a2a_pallas(send_buf)

    rcnts = recv[:, 0, 0].astype(jnp.int32)
    roffs = jnp.concatenate(
        [jnp.zeros((1,), jnp.int32), jnp.cumsum(rcnts)[:-1]]
    )
    tgt = roffs[:, None] + j[None, :]
    rvalid = j[None, :] < rcnts[:, None]
    tgt = jnp.where(rvalid, tgt, m)
    out = jnp.zeros((m + 1, N_COLS), jnp.float32)
    out = out.at[tgt.reshape(-1)].set(
        recv[:, 1:, :].reshape(-1, N_COLS).astype(jnp.float32)
    )
    return out[:m]


# device time: 20481 ns/iter; 1.9837x vs baseline; 1.9837x over previous
import jax
import jax.numpy as jnp
from jax import lax
from jax.experimental import pallas as pl
from jax.experimental.pallas import tpu as pltpu

N_DEV = 16
CAP = 63
BLK = CAP + 1
TOT = N_DEV * BLK
N_COLS = 256
M_ROWS = 512


def _iota(shape, dim):
    return lax.broadcasted_iota(jnp.int32, shape, dim)


def kernel(x, dest):
    dest_row = dest.reshape(1, M_ROWS)
    dest_col = dest.reshape(M_ROWS, 1)

    def body(x_ref, dr_ref, dc_ref, out_ref,
             send_ref, recv_ref, send_sems, recv_sems, local_sem):
        me = lax.axis_index("i")
        dr = dr_ref[...]
        dc = dc_ref[...]

        a = _iota((M_ROWS, M_ROWS), 0)
        b = _iota((M_ROWS, M_ROWS), 1)
        prior = jnp.where((dc == dr) & (a < b), 1, 0)
        r_row = jnp.sum(prior, axis=0, keepdims=True)

        d16 = _iota((N_DEV, 1), 0)
        counts = jnp.sum(
            jnp.where(d16 == dr, 1, 0), axis=1, keepdims=True
        ).astype(jnp.float32)

        row_i = _iota((TOT, 1), 0)
        d_of = row_i // BLK
        j_of = row_i % BLK - 1

        P = ((dr == d_of) & (r_row == j_of)).astype(jnp.bfloat16)
        send = jnp.dot(P, x_ref[...].astype(jnp.bfloat16),
                       preferred_element_type=jnp.float32)
        B = (d_of == _iota((TOT, N_DEV), 1)).astype(jnp.float32)
        cnt_col = jnp.dot(B, counts,
                          preferred_element_type=jnp.float32)
        send = send + jnp.where(j_of == -1, cnt_col, 0.0)
        send_ref[...] = send.astype(jnp.bfloat16)

        own = pltpu.make_async_copy(
            send_ref.at[pl.ds(me * BLK, BLK)],
            recv_ref.at[pl.ds(me * BLK, BLK)],
            local_sem,
        )
        own.start()

        sends = []
        for k in range(1, N_DEV):
            tgt = lax.rem(me + k, N_DEV)
            rdma = pltpu.make_async_remote_copy(
                src_ref=send_ref.at[pl.ds(tgt * BLK, BLK)],
                dst_ref=recv_ref.at[pl.ds(me * BLK, BLK)],
                send_sem=send_sems.at[k],
                recv_sem=recv_sems.at[k],
                device_id=(tgt,),
                device_id_type=pl.DeviceIdType.MESH,
            )
            rdma.start()
            sends.append(rdma)

        for k in range(1, N_DEV):
            src = lax.rem(me - k + N_DEV, N_DEV)
            recv = pltpu.make_async_remote_copy(
                src_ref=send_ref.at[pl.ds(src * BLK, BLK)],
                dst_ref=recv_ref.at[pl.ds(src * BLK, BLK)],
                send_sem=send_sems.at[k],
                recv_sem=recv_sems.at[k],
                device_id=(src,),
                device_id_type=pl.DeviceIdType.MESH,
            )
            recv.wait_recv()
        own.wait()

        rflat = recv_ref[...]
        S = (64 * _iota((N_DEV, TOT), 0) == _iota((N_DEV, TOT), 1)
             ).astype(jnp.float32)
        rcnts = jnp.dot(S, rflat[:, 0:1].astype(jnp.float32),
                        preferred_element_type=jnp.float32)
        B2 = (_iota((TOT, N_DEV), 1) < d_of).astype(jnp.float32)
        roffs_col = jnp.dot(B2, rcnts,
                            preferred_element_type=jnp.float32
                            ).astype(jnp.int32)
        rcnt_col = jnp.dot(B, rcnts,
                           preferred_element_type=jnp.float32
                           ).astype(jnp.int32)
        k_i = _iota((TOT, M_ROWS), 1)
        M = ((k_i == roffs_col + j_of) & (j_of >= 0) & (j_of < rcnt_col)
             ).astype(jnp.bfloat16)
        out_ref[...] = lax.dot_general(
            M, rflat,
            dimension_numbers=(((0,), (0,)), ((), ())),
            preferred_element_type=jnp.float32,
        )

        for rdma in sends:
            rdma.wait_send()

    return pl.pallas_call(
        body,
        out_shape=jax.ShapeDtypeStruct((M_ROWS, N_COLS), jnp.float32),
        in_specs=[
            pl.BlockSpec(memory_space=pltpu.VMEM),
            pl.BlockSpec(memory_space=pltpu.VMEM),
            pl.BlockSpec(memory_space=pltpu.VMEM),
        ],
        out_specs=pl.BlockSpec(memory_space=pltpu.VMEM),
        scratch_shapes=[
            pltpu.VMEM((TOT, N_COLS), jnp.bfloat16),
            pltpu.VMEM((TOT, N_COLS), jnp.bfloat16),
            pltpu.SemaphoreType.DMA((N_DEV,)),
            pltpu.SemaphoreType.DMA((N_DEV,)),
            pltpu.SemaphoreType.DMA,
        ],
    )(x, dest_row, dest_col)
